# baseline (speedup 1.0000x reference)
import functools

import jax
import jax.numpy as jnp
from jax import lax
from jax.experimental import pallas as pl
from jax.experimental.pallas import tpu as pltpu
from jax.experimental.pallas import tpu_sc as plsc

N_NODES = 10000
N_EDGES = 320000
NPAD = 10240
CHUNK = 128
NC, NS, L = 2, 16, 16
NW = NC * NS
CPW = 80
NCH = NW * CPW
EPAD = NCH * CHUNK
ROWS_PER_TILE = NPAD // NS

_MESH = plsc.VectorSubcoreMesh(
    core_axis_name="c", subcore_axis_name="s", num_cores=NC, num_subcores=NS)


_GATHER_DN = lax.GatherDimensionNumbers(
    offset_dims=(), collapsed_slice_dims=(0,), start_index_map=(0,))


def _bcast16(v, i):
    idx = jnp.full((L, 1), i, jnp.int32)
    return lax.gather(v, idx, _GATHER_DN, (1,),
                      mode=lax.GatherScatterMode.PROMISE_IN_BOUNDS)


@functools.partial(
    pl.kernel,
    out_type=[jax.ShapeDtypeStruct((NPAD,), jnp.float32),
              jax.ShapeDtypeStruct((NPAD,), jnp.float32)],
    mesh=_MESH,
    scratch_types=[
        pltpu.VMEM((CPW, CHUNK), jnp.int32),
        pltpu.VMEM((CPW, CHUNK), jnp.float32),
        pltpu.VMEM((ROWS_PER_TILE,), jnp.float32),
        pltpu.VMEM_SHARED((NPAD,), jnp.float32),
    ],
)
def _deg_kernel(dst_hbm, w_hbm, out0_hbm, out1_hbm, dst_v, w_v, zb, shared):
    c = lax.axis_index("c")
    s = lax.axis_index("s")
    wid = c * NS + s

    def zero_body(i, _):
        zb[pl.ds(i * L, L)] = jnp.zeros((L,), jnp.float32)
        return 0

    lax.fori_loop(0, ROWS_PER_TILE // L, zero_body, 0)
    pltpu.sync_copy(zb, shared.at[pl.ds(s * ROWS_PER_TILE, ROWS_PER_TILE)])
    plsc.subcore_barrier()

    pltpu.sync_copy(dst_hbm.at[pl.ds(wid * CPW, CPW)], dst_v)
    pltpu.sync_copy(w_hbm.at[pl.ds(wid * CPW, CPW)], w_v)

    def chunk_body(j, _):
        pltpu.sync_copy(w_v.at[j], shared.at[dst_v.at[j]], add=True)
        return 0

    lax.fori_loop(0, CPW, chunk_body, 0)
    plsc.subcore_barrier()
    sl = pl.ds(s * ROWS_PER_TILE, ROWS_PER_TILE)

    @pl.when(c == 0)
    def _():
        pltpu.sync_copy(shared.at[sl], out0_hbm.at[sl])

    @pl.when(c == 1)
    def _():
        pltpu.sync_copy(shared.at[sl], out1_hbm.at[sl])


def _make_agg(D, NB, packed=False):
    TW = D // 2 if packed else D
    tdt = jnp.int32 if packed else jnp.float32
    @functools.partial(
        pl.kernel,
        out_type=[jax.ShapeDtypeStruct((NPAD, D), jnp.float32),
                  jax.ShapeDtypeStruct((NPAD, D), jnp.float32)],
        mesh=_MESH,
        scratch_types=[
            pltpu.VMEM((CPW, CHUNK), jnp.int32),
            pltpu.VMEM((CPW, CHUNK), jnp.int32),
            pltpu.VMEM((CPW * CHUNK,), jnp.float32),
            [pltpu.VMEM((CHUNK, TW), tdt) for _ in range(NB)],
            [pltpu.VMEM((CHUNK, D), jnp.float32) for _ in range(NB)],
            pltpu.VMEM_SHARED((NPAD, D), jnp.float32),
            [pltpu.SemaphoreType.DMA for _ in range(NB)],
            [pltpu.SemaphoreType.DMA for _ in range(NB)],
        ],
        compiler_params=pltpu.CompilerParams(use_tc_tiling_on_sc=False,
                                             needs_layout_passes=False),
    )
    def agg(hs_hbm, src_hbm, dst_hbm, wf_hbm, out0_hbm, out1_hbm,
            src_v, dst_v, w_v, gb, sb, shared, gsem, ssem):
        c = lax.axis_index("c")
        s = lax.axis_index("s")
        wid = c * NS + s

        for r in range(CHUNK):
            for f in range(D // L):
                sb[0][r, pl.ds(f * L, L)] = jnp.zeros((L,), jnp.float32)
        for t in range(ROWS_PER_TILE // CHUNK):
            pltpu.sync_copy(
                sb[0], shared.at[pl.ds(s * ROWS_PER_TILE + t * CHUNK, CHUNK)])
        plsc.subcore_barrier()

        pltpu.sync_copy(src_hbm.at[pl.ds(wid * CPW, CPW)], src_v)
        pltpu.sync_copy(dst_hbm.at[pl.ds(wid * CPW, CPW)], dst_v)
        pltpu.sync_copy(wf_hbm.at[pl.ds(wid * CPW * CHUNK, CPW * CHUNK)], w_v)

        def scale(gbuf, sbuf, j):
            for g in range(CHUNK // L):
                w16 = w_v[pl.ds(j * CHUNK + g * L, L)]
                for i in range(L):
                    wb = _bcast16(w16, i)
                    r = g * L + i
                    if packed:
                        for f in range(D // 32):
                            v = gbuf[r, pl.ds(f * L, L)]
                            fe = plsc.bitcast(v << 16, jnp.float32)
                            fo = plsc.bitcast(v & jnp.int32(-65536),
                                              jnp.float32)
                            ev = lax.iota(jnp.int32, L) * 2 + 32 * f
                            rr = jnp.full((L,), r, jnp.int32)
                            plsc.store_scatter(sbuf, [rr, ev], fe * wb)
                            plsc.store_scatter(sbuf, [rr, ev + 1], fo * wb)
                    else:
                        for f in range(D // L):
                            sl = pl.ds(f * L, L)
                            sbuf[r, sl] = gbuf[r, sl] * wb

        for b in range(NB):
            pltpu.async_copy(hs_hbm.at[src_v.at[b]], gb[b], gsem[b])

        def quad_body(jj, _):
            scs = []
            for b in range(NB):
                j = jj * NB + b
                pltpu.make_async_copy(
                    hs_hbm.at[src_v.at[j]], gb[b], gsem[b]).wait()
                scale(gb[b], sb[b], j)
                scs.append(pltpu.async_copy(
                    sb[b], shared.at[dst_v.at[j]], ssem[b], add=True))

                @pl.when(j + NB < CPW)
                def _():
                    pltpu.async_copy(hs_hbm.at[src_v.at[j + NB]],
                                     gb[b], gsem[b])
            for sc in scs:
                sc.wait()
            return 0

        lax.fori_loop(0, CPW // NB, quad_body, 0)
        plsc.subcore_barrier()
        sl = pl.ds(s * ROWS_PER_TILE, ROWS_PER_TILE)

        @pl.when(c == 0)
        def _():
            pltpu.sync_copy(shared.at[sl], out0_hbm.at[sl])

        @pl.when(c == 1)
        def _():
            pltpu.sync_copy(shared.at[sl], out1_hbm.at[sl])

    return agg


_agg64 = _make_agg(64, 2, packed=True)
_agg16 = _make_agg(16, 4)

_RB = 1280
_GRID = NPAD // _RB


def _k2_body(x_ref, w1_ref, dp0_ref, dp1_ref, dis_ref, hs1_ref, sc1_ref):
    deg = 1.0 + dp0_ref[...] + dp1_ref[...]
    dis = lax.rsqrt(deg)
    dis_ref[...] = dis
    h = jnp.dot(x_ref[...], w1_ref[...], preferred_element_type=jnp.float32)
    hs = h * dis
    hs1_ref[...] = hs
    sc1_ref[...] = hs * dis


def _k4_body(p0_ref, p1_ref, sc1_ref, dis_ref, b1_ref, w2_ref,
             xemb_ref, hs2_ref, sc2_ref):
    dis = dis_ref[...]
    out1 = dis * (p0_ref[...] + p1_ref[...]) + sc1_ref[...] + b1_ref[...]
    xemb_ref[...] = out1
    h = jnp.maximum(out1, 0.0)
    h2 = jnp.dot(h, w2_ref[...], preferred_element_type=jnp.float32)
    hs2 = h2 * dis
    hs2_ref[...] = hs2
    sc2_ref[...] = hs2 * dis


def _k6_body(p0_ref, p1_ref, sc2_ref, dis_ref, b2_ref, out_ref):
    out_ref[...] = (dis_ref[...] * (p0_ref[...] + p1_ref[...])
                    + sc2_ref[...] + b2_ref[...])


def _row_spec(d):
    return pl.BlockSpec((_RB, d), lambda i: (i, 0))


def _full_spec(shape):
    return pl.BlockSpec(shape, lambda i: (0, 0))


_k2 = pl.pallas_call(
    _k2_body,
    grid=(_GRID,),
    in_specs=[_row_spec(128), _full_spec((128, 64)), _row_spec(1), _row_spec(1)],
    out_specs=[_row_spec(1), _row_spec(64), _row_spec(64)],
    out_shape=[jax.ShapeDtypeStruct((NPAD, 1), jnp.float32),
               jax.ShapeDtypeStruct((NPAD, 64), jnp.float32),
               jax.ShapeDtypeStruct((NPAD, 64), jnp.float32)],
)

_k4 = pl.pallas_call(
    _k4_body,
    grid=(_GRID,),
    in_specs=[_row_spec(64), _row_spec(64), _row_spec(64), _row_spec(1),
              _full_spec((1, 64)), _full_spec((64, 16))],
    out_specs=[_row_spec(64), _row_spec(16), _row_spec(16)],
    out_shape=[jax.ShapeDtypeStruct((NPAD, 64), jnp.float32),
               jax.ShapeDtypeStruct((NPAD, 16), jnp.float32),
               jax.ShapeDtypeStruct((NPAD, 16), jnp.float32)],
)

_k6 = pl.pallas_call(
    _k6_body,
    grid=(_GRID,),
    in_specs=[_row_spec(16), _row_spec(16), _row_spec(16), _row_spec(1),
              _full_spec((1, 16))],
    out_specs=_row_spec(16),
    out_shape=jax.ShapeDtypeStruct((NPAD, 16), jnp.float32),
)


def kernel(x, edge_index, edge_weight, W1, b1, W2, b2):
    src = edge_index[0].astype(jnp.int32)
    dst = edge_index[1].astype(jnp.int32)
    w = edge_weight.astype(jnp.float32)

    pe = EPAD - N_EDGES
    src_p = jnp.concatenate([src, jnp.zeros((pe,), jnp.int32)]).reshape(NCH, CHUNK)
    dst_p = jnp.concatenate([dst, jnp.zeros((pe,), jnp.int32)]).reshape(NCH, CHUNK)
    w_p = jnp.concatenate([w, jnp.zeros((pe,), jnp.float32)]).reshape(NCH, CHUNK)
    x_p = jnp.pad(x, ((0, NPAD - N_NODES), (0, 0)))

    dp0, dp1 = _deg_kernel(dst_p, w_p)
    dis, hs1, sc1 = _k2(x_p, W1, dp0.reshape(NPAD, 1), dp1.reshape(NPAD, 1))
    hs1_packed = jax.lax.bitcast_convert_type(
        hs1.astype(jnp.bfloat16).reshape(NPAD, 32, 2), jnp.int32)
    w_flat = w_p.reshape(EPAD)
    a1p0, a1p1 = _agg64(hs1_packed, src_p, dst_p, w_flat)
    xemb, hs2, sc2 = _k4(a1p0, a1p1, sc1, dis, b1.reshape(1, 64), W2)
    a2p0, a2p1 = _agg16(hs2, src_p, dst_p, w_flat)
    out2 = _k6(a2p0, a2p1, sc2, dis, b2.reshape(1, 16))
    return out2[:N_NODES], xemb[:N_NODES]


# speedup vs baseline: 2.2473x; 2.2473x over previous
import functools

import jax
import jax.numpy as jnp
from jax import lax
from jax.experimental import pallas as pl
from jax.experimental.pallas import tpu as pltpu
from jax.experimental.pallas import tpu_sc as plsc

N_NODES = 10000
N_EDGES = 320000
NPAD = 10240
CHUNK = 128
NC, NS, L = 2, 16, 16
NW = NC * NS
EPT = N_EDGES // NW
FC = EPT // CHUNK
TAIL = EPT - FC * CHUNK
ROWS_PER_TILE = NPAD // NS

_MESH = plsc.VectorSubcoreMesh(
    core_axis_name="c", subcore_axis_name="s", num_cores=NC, num_subcores=NS)

_SC_PARAMS = pltpu.CompilerParams(use_tc_tiling_on_sc=False,
                                  needs_layout_passes=False)

_GATHER_DN = lax.GatherDimensionNumbers(
    offset_dims=(), collapsed_slice_dims=(0,), start_index_map=(0,))


def _bcast16(v, i):
    idx = jnp.full((L, 1), i, jnp.int32)
    return lax.gather(v, idx, _GATHER_DN, (1,),
                      mode=lax.GatherScatterMode.PROMISE_IN_BOUNDS)


@functools.partial(
    pl.kernel,
    out_type=[jax.ShapeDtypeStruct((NPAD,), jnp.float32),
              jax.ShapeDtypeStruct((NPAD,), jnp.float32)],
    mesh=_MESH,
    scratch_types=[
        pltpu.VMEM((EPT,), jnp.int32),
        pltpu.VMEM((EPT,), jnp.float32),
        pltpu.VMEM((ROWS_PER_TILE,), jnp.float32),
        pltpu.VMEM_SHARED((NPAD,), jnp.float32),
    ],
    compiler_params=_SC_PARAMS,
)
def _deg_kernel(ei_hbm, w_hbm, out0_hbm, out1_hbm, dst_v, w_v, zb, shared):
    c = lax.axis_index("c")
    s = lax.axis_index("s")
    wid = c * NS + s

    def zero_body(i, _):
        zb[pl.ds(i * L, L)] = jnp.zeros((L,), jnp.float32)
        return 0

    lax.fori_loop(0, ROWS_PER_TILE // L, zero_body, 0)
    pltpu.sync_copy(zb, shared.at[pl.ds(s * ROWS_PER_TILE, ROWS_PER_TILE)])
    plsc.subcore_barrier()

    pltpu.sync_copy(ei_hbm.at[1, pl.ds(wid * EPT, EPT)], dst_v)
    pltpu.sync_copy(w_hbm.at[pl.ds(wid * EPT, EPT)], w_v)

    def chunk_body(j, _):
        sl = pl.ds(j * CHUNK, CHUNK)
        pltpu.sync_copy(w_v.at[sl], shared.at[dst_v.at[sl]], add=True)
        return 0

    lax.fori_loop(0, FC, chunk_body, 0)
    tl = pl.ds(FC * CHUNK, TAIL)
    pltpu.sync_copy(w_v.at[tl], shared.at[dst_v.at[tl]], add=True)
    plsc.subcore_barrier()
    sl = pl.ds(s * ROWS_PER_TILE, ROWS_PER_TILE)

    @pl.when(c == 0)
    def _():
        pltpu.sync_copy(shared.at[sl], out0_hbm.at[sl])

    @pl.when(c == 1)
    def _():
        pltpu.sync_copy(shared.at[sl], out1_hbm.at[sl])


def _make_agg(D, NB, packed=False):
    TW = D // 2 if packed else D
    tdt = jnp.int32 if packed else jnp.float32

    @functools.partial(
        pl.kernel,
        out_type=[jax.ShapeDtypeStruct((NPAD, D), jnp.float32),
                  jax.ShapeDtypeStruct((NPAD, D), jnp.float32)],
        mesh=_MESH,
        scratch_types=[
            pltpu.VMEM((EPT,), jnp.int32),
            pltpu.VMEM((EPT,), jnp.int32),
            pltpu.VMEM((EPT,), jnp.float32),
            [pltpu.VMEM((CHUNK, TW), tdt) for _ in range(NB)],
            [pltpu.VMEM((CHUNK, D), jnp.float32) for _ in range(NB)],
            pltpu.VMEM_SHARED((NPAD, D), jnp.float32),
            [pltpu.SemaphoreType.DMA for _ in range(NB)],
            [pltpu.SemaphoreType.DMA for _ in range(NB)],
        ],
        compiler_params=_SC_PARAMS,
    )
    def agg(hs_hbm, ei_hbm, wf_hbm, out0_hbm, out1_hbm,
            src_v, dst_v, w_v, gb, sb, shared, gsem, ssem):
        c = lax.axis_index("c")
        s = lax.axis_index("s")
        wid = c * NS + s

        for r in range(CHUNK):
            for f in range(D // L):
                sb[0][r, pl.ds(f * L, L)] = jnp.zeros((L,), jnp.float32)
        for t in range(ROWS_PER_TILE // CHUNK):
            pltpu.sync_copy(
                sb[0], shared.at[pl.ds(s * ROWS_PER_TILE + t * CHUNK, CHUNK)])
        plsc.subcore_barrier()

        pltpu.sync_copy(ei_hbm.at[0, pl.ds(wid * EPT, EPT)], src_v)
        pltpu.sync_copy(ei_hbm.at[1, pl.ds(wid * EPT, EPT)], dst_v)
        pltpu.sync_copy(wf_hbm.at[pl.ds(wid * EPT, EPT)], w_v)

        def scale(gbuf, sbuf, j, rows):
            for g in range(rows // L):
                w16 = w_v[pl.ds(j * CHUNK + g * L, L)]
                for i in range(L):
                    wb = _bcast16(w16, i)
                    r = g * L + i
                    if packed:
                        for f in range(D // 32):
                            v = gbuf[r, pl.ds(f * L, L)]
                            fe = plsc.bitcast(v << 16, jnp.float32)
                            fo = plsc.bitcast(v & jnp.int32(-65536),
                                              jnp.float32)
                            sbuf[r, pl.ds(32 * f, L)] = fe * wb
                            sbuf[r, pl.ds(32 * f + 16, L)] = fo * wb
                    else:
                        for f in range(D // L):
                            sl = pl.ds(f * L, L)
                            sbuf[r, sl] = gbuf[r, sl] * wb

        for b in range(NB):
            pltpu.async_copy(
                hs_hbm.at[src_v.at[pl.ds(b * CHUNK, CHUNK)]], gb[b], gsem[b])

        def pipe_body(jj, _):
            scs = []
            for b in range(NB):
                j = jj * NB + b
                sl = pl.ds(j * CHUNK, CHUNK)
                pltpu.make_async_copy(
                    hs_hbm.at[src_v.at[sl]], gb[b], gsem[b]).wait()
                scale(gb[b], sb[b], j, CHUNK)
                scs.append(pltpu.async_copy(
                    sb[b], shared.at[dst_v.at[sl]], ssem[b], add=True))

                @pl.when(j + NB < FC)
                def _():
                    nsl = pl.ds((j + NB) * CHUNK, CHUNK)
                    pltpu.async_copy(hs_hbm.at[src_v.at[nsl]], gb[b], gsem[b])
            for sc in scs:
                sc.wait()
            return 0

        lax.fori_loop(0, FC // NB, pipe_body, 0)
        tl = pl.ds(FC * CHUNK, TAIL)
        gt = gb[0].at[pl.ds(0, TAIL)]
        pltpu.async_copy(hs_hbm.at[src_v.at[tl]], gt, gsem[0]).wait()
        scale(gb[0], sb[0], FC, TAIL)
        pltpu.sync_copy(sb[0].at[pl.ds(0, TAIL)], shared.at[dst_v.at[tl]],
                        add=True)
        plsc.subcore_barrier()
        sl = pl.ds(s * ROWS_PER_TILE, ROWS_PER_TILE)

        @pl.when(c == 0)
        def _():
            pltpu.sync_copy(shared.at[sl], out0_hbm.at[sl])

        @pl.when(c == 1)
        def _():
            pltpu.sync_copy(shared.at[sl], out1_hbm.at[sl])

    return agg


_agg64 = _make_agg(64, 2, packed=True)
_agg16 = _make_agg(16, 2)

_RB = 1280
_GRID = NPAD // _RB


def _k2_body(x_ref, w1_ref, dp0_ref, dp1_ref, dis_ref, hs1_ref, sc1_ref):
    deg = 1.0 + dp0_ref[...] + dp1_ref[...]
    dis = lax.rsqrt(deg)
    dis_ref[...] = dis
    h = jnp.dot(x_ref[...], w1_ref[...], preferred_element_type=jnp.float32)
    hs = h * dis
    hs1_ref[...] = hs
    sc1_ref[...] = hs * dis


def _k4_body(p0_ref, p1_ref, sc1_ref, dis_ref, b1_ref, w2_ref,
             xemb_ref, hs2_ref, sc2_ref):
    dis = dis_ref[...]
    out1 = dis * (p0_ref[...] + p1_ref[...]) + sc1_ref[...] + b1_ref[...]
    xemb_ref[...] = out1
    h = jnp.maximum(out1, 0.0)
    h2 = jnp.dot(h, w2_ref[...], preferred_element_type=jnp.float32)
    hs2 = h2 * dis
    hs2_ref[...] = hs2
    sc2_ref[...] = hs2 * dis


def _k6_body(p0_ref, p1_ref, sc2_ref, dis_ref, b2_ref, out_ref):
    out_ref[...] = (dis_ref[...] * (p0_ref[...] + p1_ref[...])
                    + sc2_ref[...] + b2_ref[...])


def _row_spec(d):
    return pl.BlockSpec((_RB, d), lambda i: (i, 0))


def _full_spec(shape):
    return pl.BlockSpec(shape, lambda i: (0, 0))


_k2 = pl.pallas_call(
    _k2_body,
    grid=(_GRID,),
    in_specs=[_row_spec(128), _full_spec((128, 64)), _row_spec(1), _row_spec(1)],
    out_specs=[_row_spec(1), _row_spec(64), _row_spec(64)],
    out_shape=[jax.ShapeDtypeStruct((NPAD, 1), jnp.float32),
               jax.ShapeDtypeStruct((NPAD, 64), jnp.float32),
               jax.ShapeDtypeStruct((NPAD, 64), jnp.float32)],
)

_k4 = pl.pallas_call(
    _k4_body,
    grid=(_GRID,),
    in_specs=[_row_spec(64), _row_spec(64), _row_spec(64), _row_spec(1),
              _full_spec((1, 64)), _full_spec((64, 16))],
    out_specs=[_row_spec(64), _row_spec(16), _row_spec(16)],
    out_shape=[jax.ShapeDtypeStruct((NPAD, 64), jnp.float32),
               jax.ShapeDtypeStruct((NPAD, 16), jnp.float32),
               jax.ShapeDtypeStruct((NPAD, 16), jnp.float32)],
)

_k6 = pl.pallas_call(
    _k6_body,
    grid=(_GRID,),
    in_specs=[_row_spec(16), _row_spec(16), _row_spec(16), _row_spec(1),
              _full_spec((1, 16))],
    out_specs=_row_spec(16),
    out_shape=jax.ShapeDtypeStruct((NPAD, 16), jnp.float32),
)


def kernel(x, edge_index, edge_weight, W1, b1, W2, b2):
    ei = edge_index.astype(jnp.int32)
    w = edge_weight.astype(jnp.float32)
    x_p = jnp.pad(x, ((0, NPAD - N_NODES), (0, 0)))

    dp0, dp1 = _deg_kernel(ei, w)
    dis, hs1, sc1 = _k2(x_p, W1, dp0.reshape(NPAD, 1), dp1.reshape(NPAD, 1))
    hs1_packed = jax.lax.bitcast_convert_type(
        hs1.astype(jnp.bfloat16).reshape(NPAD, 2, 2, 16)
        .transpose(0, 1, 3, 2).reshape(NPAD, 32, 2), jnp.int32)
    a1p0, a1p1 = _agg64(hs1_packed, ei, w)
    xemb, hs2, sc2 = _k4(a1p0, a1p1, sc1, dis, b1.reshape(1, 64), W2)
    a2p0, a2p1 = _agg16(hs2, ei, w)
    out2 = _k6(a2p0, a2p1, sc2, dis, b2.reshape(1, 16))
    return out2[:N_NODES], xemb[:N_NODES]


# speedup vs baseline: 2.2546x; 1.0033x over previous
import functools

import jax
import jax.numpy as jnp
from jax import lax
from jax.experimental import pallas as pl
from jax.experimental.pallas import tpu as pltpu
from jax.experimental.pallas import tpu_sc as plsc

N_NODES = 10000
N_EDGES = 320000
NPAD = 10240
CHUNK = 128
NC, NS, L = 2, 16, 16
NW = NC * NS
EPT = N_EDGES // NW
FC = EPT // CHUNK
TAIL = EPT - FC * CHUNK
ROWS_PER_TILE = NPAD // NS

_MESH = plsc.VectorSubcoreMesh(
    core_axis_name="c", subcore_axis_name="s", num_cores=NC, num_subcores=NS)

_SC_PARAMS = pltpu.CompilerParams(use_tc_tiling_on_sc=False,
                                  needs_layout_passes=False)

_GATHER_DN = lax.GatherDimensionNumbers(
    offset_dims=(), collapsed_slice_dims=(0,), start_index_map=(0,))


def _bcast16(v, i):
    idx = jnp.full((L, 1), i, jnp.int32)
    return lax.gather(v, idx, _GATHER_DN, (1,),
                      mode=lax.GatherScatterMode.PROMISE_IN_BOUNDS)


@functools.partial(
    pl.kernel,
    out_type=[jax.ShapeDtypeStruct((NPAD,), jnp.float32),
              jax.ShapeDtypeStruct((NPAD,), jnp.float32)],
    mesh=_MESH,
    scratch_types=[
        pltpu.VMEM((EPT,), jnp.int32),
        pltpu.VMEM((EPT,), jnp.float32),
        pltpu.VMEM((ROWS_PER_TILE,), jnp.float32),
        pltpu.VMEM_SHARED((NPAD,), jnp.float32),
    ],
    compiler_params=_SC_PARAMS,
)
def _deg_kernel(ei_hbm, w_hbm, out0_hbm, out1_hbm, dst_v, w_v, zb, shared):
    c = lax.axis_index("c")
    s = lax.axis_index("s")
    wid = c * NS + s

    def zero_body(i, _):
        zb[pl.ds(i * L, L)] = jnp.zeros((L,), jnp.float32)
        return 0

    lax.fori_loop(0, ROWS_PER_TILE // L, zero_body, 0)
    pltpu.sync_copy(zb, shared.at[pl.ds(s * ROWS_PER_TILE, ROWS_PER_TILE)])
    plsc.subcore_barrier()

    pltpu.sync_copy(ei_hbm.at[1, pl.ds(wid * EPT, EPT)], dst_v)
    pltpu.sync_copy(w_hbm.at[pl.ds(wid * EPT, EPT)], w_v)

    def chunk_body(j, _):
        sl = pl.ds(j * CHUNK, CHUNK)
        pltpu.sync_copy(w_v.at[sl], shared.at[dst_v.at[sl]], add=True)
        return 0

    lax.fori_loop(0, FC, chunk_body, 0)
    tl = pl.ds(FC * CHUNK, TAIL)
    pltpu.sync_copy(w_v.at[tl], shared.at[dst_v.at[tl]], add=True)
    plsc.subcore_barrier()
    sl = pl.ds(s * ROWS_PER_TILE, ROWS_PER_TILE)

    @pl.when(c == 0)
    def _():
        pltpu.sync_copy(shared.at[sl], out0_hbm.at[sl])

    @pl.when(c == 1)
    def _():
        pltpu.sync_copy(shared.at[sl], out1_hbm.at[sl])


def _make_agg(D, NB, packed=False):
    TW = D // 2 if packed else D
    tdt = jnp.int32 if packed else jnp.float32

    @functools.partial(
        pl.kernel,
        out_type=[jax.ShapeDtypeStruct((NPAD, D), jnp.float32),
                  jax.ShapeDtypeStruct((NPAD, D), jnp.float32)],
        mesh=_MESH,
        scratch_types=[
            pltpu.VMEM((EPT,), jnp.int32),
            pltpu.VMEM((EPT,), jnp.int32),
            pltpu.VMEM((EPT,), jnp.float32),
            [pltpu.VMEM((CHUNK, TW), tdt) for _ in range(NB)],
            [pltpu.VMEM((CHUNK, D), jnp.float32) for _ in range(NB)],
            pltpu.VMEM_SHARED((NPAD, D), jnp.float32),
            [pltpu.SemaphoreType.DMA for _ in range(NB)],
            [pltpu.SemaphoreType.DMA for _ in range(NB)],
        ],
        compiler_params=_SC_PARAMS,
    )
    def agg(hs_hbm, ei_hbm, wf_hbm, out0_hbm, out1_hbm,
            src_v, dst_v, w_v, gb, sb, shared, gsem, ssem):
        c = lax.axis_index("c")
        s = lax.axis_index("s")
        wid = c * NS + s

        for r in range(CHUNK):
            for f in range(D // L):
                sb[0][r, pl.ds(f * L, L)] = jnp.zeros((L,), jnp.float32)
        for t in range(ROWS_PER_TILE // CHUNK):
            pltpu.sync_copy(
                sb[0], shared.at[pl.ds(s * ROWS_PER_TILE + t * CHUNK, CHUNK)])
        plsc.subcore_barrier()

        pltpu.sync_copy(ei_hbm.at[0, pl.ds(wid * EPT, EPT)], src_v)
        pltpu.sync_copy(ei_hbm.at[1, pl.ds(wid * EPT, EPT)], dst_v)
        pltpu.sync_copy(wf_hbm.at[pl.ds(wid * EPT, EPT)], w_v)

        def scale(gbuf, sbuf, j, rows):
            for g in range(rows // L):
                w16 = w_v[pl.ds(j * CHUNK + g * L, L)]
                for i in range(L):
                    wb = _bcast16(w16, i)
                    r = g * L + i
                    if packed:
                        for f in range(D // 32):
                            v = gbuf[r, pl.ds(f * L, L)]
                            fe = plsc.bitcast(v << 16, jnp.float32)
                            fo = plsc.bitcast(v & jnp.int32(-65536),
                                              jnp.float32)
                            sbuf[r, pl.ds(32 * f, L)] = fe * wb
                            sbuf[r, pl.ds(32 * f + 16, L)] = fo * wb
                    else:
                        for f in range(D // L):
                            sl = pl.ds(f * L, L)
                            sbuf[r, sl] = gbuf[r, sl] * wb

        for b in range(NB):
            pltpu.async_copy(
                hs_hbm.at[src_v.at[pl.ds(b * CHUNK, CHUNK)]], gb[b], gsem[b])

        def pipe_body(jj, _):
            scs = []
            for b in range(NB):
                j = jj * NB + b
                sl = pl.ds(j * CHUNK, CHUNK)
                pltpu.make_async_copy(
                    hs_hbm.at[src_v.at[sl]], gb[b], gsem[b]).wait()
                scale(gb[b], sb[b], j, CHUNK)
                scs.append(pltpu.async_copy(
                    sb[b], shared.at[dst_v.at[sl]], ssem[b], add=True))

                @pl.when(j + NB < FC)
                def _():
                    nsl = pl.ds((j + NB) * CHUNK, CHUNK)
                    pltpu.async_copy(hs_hbm.at[src_v.at[nsl]], gb[b], gsem[b])
            for sc in scs:
                sc.wait()
            return 0

        lax.fori_loop(0, FC // NB, pipe_body, 0)
        tl = pl.ds(FC * CHUNK, TAIL)
        gt = gb[0].at[pl.ds(0, TAIL)]
        pltpu.async_copy(hs_hbm.at[src_v.at[tl]], gt, gsem[0]).wait()
        scale(gb[0], sb[0], FC, TAIL)
        pltpu.sync_copy(sb[0].at[pl.ds(0, TAIL)], shared.at[dst_v.at[tl]],
                        add=True)
        plsc.subcore_barrier()
        sl = pl.ds(s * ROWS_PER_TILE, ROWS_PER_TILE)

        @pl.when(c == 0)
        def _():
            pltpu.sync_copy(shared.at[sl], out0_hbm.at[sl])

        @pl.when(c == 1)
        def _():
            pltpu.sync_copy(shared.at[sl], out1_hbm.at[sl])

    return agg


_agg64 = _make_agg(64, 2, packed=True)
_agg16 = _make_agg(16, 2)

_RB = 1280
_GRID = NPAD // _RB


def _k2_body(x_ref, w1_ref, dp0_ref, dp1_ref, dis_ref, hs1_ref, sc1_ref):
    deg = 1.0 + dp0_ref[...] + dp1_ref[...]
    dis = lax.rsqrt(deg)
    dis_ref[...] = dis
    h = jnp.dot(x_ref[...], w1_ref[...], preferred_element_type=jnp.float32)
    hs = h * dis
    hs1_ref[...] = hs.astype(jnp.bfloat16)
    sc1_ref[...] = hs * dis


def _k4_body(p0_ref, p1_ref, sc1_ref, dis_ref, b1_ref, w2_ref,
             xemb_ref, hs2_ref, sc2_ref):
    dis = dis_ref[...]
    out1 = dis * (p0_ref[...] + p1_ref[...]) + sc1_ref[...] + b1_ref[...]
    xemb_ref[...] = out1
    h = jnp.maximum(out1, 0.0)
    h2 = jnp.dot(h, w2_ref[...], preferred_element_type=jnp.float32)
    hs2 = h2 * dis
    hs2_ref[...] = hs2
    sc2_ref[...] = hs2 * dis


def _k6_body(p0_ref, p1_ref, sc2_ref, dis_ref, b2_ref, out_ref):
    out_ref[...] = (dis_ref[...] * (p0_ref[...] + p1_ref[...])
                    + sc2_ref[...] + b2_ref[...])


def _row_spec(d):
    return pl.BlockSpec((_RB, d), lambda i: (i, 0))


def _full_spec(shape):
    return pl.BlockSpec(shape, lambda i: (0, 0))


_k2 = pl.pallas_call(
    _k2_body,
    grid=(_GRID,),
    in_specs=[_row_spec(128), _full_spec((128, 64)), _row_spec(1), _row_spec(1)],
    out_specs=[_row_spec(1), _row_spec(64), _row_spec(64)],
    out_shape=[jax.ShapeDtypeStruct((NPAD, 1), jnp.float32),
               jax.ShapeDtypeStruct((NPAD, 64), jnp.bfloat16),
               jax.ShapeDtypeStruct((NPAD, 64), jnp.float32)],
)

_k4 = pl.pallas_call(
    _k4_body,
    grid=(_GRID,),
    in_specs=[_row_spec(64), _row_spec(64), _row_spec(64), _row_spec(1),
              _full_spec((1, 64)), _full_spec((64, 16))],
    out_specs=[_row_spec(64), _row_spec(16), _row_spec(16)],
    out_shape=[jax.ShapeDtypeStruct((NPAD, 64), jnp.float32),
               jax.ShapeDtypeStruct((NPAD, 16), jnp.float32),
               jax.ShapeDtypeStruct((NPAD, 16), jnp.float32)],
)

_k6 = pl.pallas_call(
    _k6_body,
    grid=(_GRID,),
    in_specs=[_row_spec(16), _row_spec(16), _row_spec(16), _row_spec(1),
              _full_spec((1, 16))],
    out_specs=_row_spec(16),
    out_shape=jax.ShapeDtypeStruct((NPAD, 16), jnp.float32),
)


def kernel(x, edge_index, edge_weight, W1, b1, W2, b2):
    ei = edge_index.astype(jnp.int32)
    w = edge_weight.astype(jnp.float32)
    x_p = jnp.pad(x, ((0, NPAD - N_NODES), (0, 0)))

    dp0, dp1 = _deg_kernel(ei, w)
    dis, hs1, sc1 = _k2(x_p, W1, dp0.reshape(NPAD, 1), dp1.reshape(NPAD, 1))
    hs1_packed = jax.lax.bitcast_convert_type(
        hs1.reshape(NPAD, 2, 2, 16)
        .transpose(0, 1, 3, 2).reshape(NPAD, 32, 2), jnp.int32)
    a1p0, a1p1 = _agg64(hs1_packed, ei, w)
    xemb, hs2, sc2 = _k4(a1p0, a1p1, sc1, dis, b1.reshape(1, 64), W2)
    a2p0, a2p1 = _agg16(hs2, ei, w)
    out2 = _k6(a2p0, a2p1, sc2, dis, b2.reshape(1, 16))
    return out2[:N_NODES], xemb[:N_NODES]
